# Initial kernel scaffold; baseline (speedup 1.0000x reference)
import jax
import jax.numpy as jnp
from jax.experimental import pallas as pl


def kernel(h, edge_index):
    raise NotImplementedError("write your pallas kernel here")


# speedup vs baseline: 203.5958x; 203.5958x over previous
import functools

import jax
import jax.numpy as jnp
from jax import lax
from jax.experimental import pallas as pl
from jax.experimental.pallas import tpu as pltpu
from jax.experimental.pallas import tpu_sc as plsc

N = 100000
E = 6400000
L = 3

LANES = 16
TILES = 16
NP = 102400
EPT = E // TILES
CH = 16000
CHUNKS = EPT // CH
SLICE = NP // TILES
VPT = SLICE // LANES


def _rsqrt(x):
  i = lax.bitcast_convert_type(x, jnp.int32)
  i = jnp.int32(0x5F3759DF) - lax.shift_right_logical(i, 1)
  y = lax.bitcast_convert_type(i, jnp.float32)
  for _ in range(3):
    y = y * (1.5 - 0.5 * x * y * y)
  return y


def _sc_body(h_hbm, src_hbm, dst_hbm, out_hbm,
             deg_o_sp, deg_i_sp, p_sp, agg_sp, c_sp,
             src_buf, dst_buf, val_buf, ones_buf,
             agg_loc, na, nb, np_, cbuf, red_buf):
  cid = lax.axis_index("c")
  sid = lax.axis_index("s")

  @pl.when(cid == 0)
  def _core0():
    ebase = sid * EPT
    nbase = sid * SLICE
    zeros16 = jnp.zeros((LANES,), jnp.float32)
    ones16 = jnp.ones((LANES,), jnp.float32)

    def fill_ones(i, _):
      ones_buf[pl.ds(i * LANES, LANES)] = ones16
      return 0
    lax.fori_loop(0, CH // LANES, fill_ones, 0)

    def fill_zero(i, _):
      agg_loc[pl.ds(i * LANES, LANES)] = zeros16
      return 0
    lax.fori_loop(0, VPT, fill_zero, 0)

    pltpu.sync_copy(agg_loc, deg_o_sp.at[pl.ds(nbase, SLICE)])
    pltpu.sync_copy(agg_loc, deg_i_sp.at[pl.ds(nbase, SLICE)])
    pltpu.sync_copy(agg_loc, agg_sp.at[pl.ds(nbase, SLICE)])
    pltpu.sync_copy(agg_loc, p_sp.at[pl.ds(nbase, SLICE)])
    plsc.subcore_barrier()

    def deg_chunk(g, _):
      off = ebase + g * CH
      pltpu.sync_copy(src_hbm.at[pl.ds(off, CH)], src_buf)
      pltpu.sync_copy(ones_buf, deg_o_sp.at[src_buf], add=True)
      pltpu.sync_copy(dst_hbm.at[pl.ds(off, CH)], dst_buf)
      pltpu.sync_copy(ones_buf, deg_i_sp.at[dst_buf], add=True)
      return 0
    lax.fori_loop(0, CHUNKS, deg_chunk, 0)
    plsc.subcore_barrier()

    pltpu.sync_copy(h_hbm.at[pl.ds(nbase, SLICE)], na)
    pltpu.sync_copy(deg_o_sp.at[pl.ds(nbase, SLICE)], nb)

    def prep_o(i, _):
      s = pl.ds(i * LANES, LANES)
      no = _rsqrt(jnp.maximum(nb[s], 1.0))
      nb[s] = no
      np_[s] = na[s] * no
      return 0
    lax.fori_loop(0, VPT, prep_o, 0)
    pltpu.sync_copy(nb, deg_o_sp.at[pl.ds(nbase, SLICE)])
    pltpu.sync_copy(np_, p_sp.at[pl.ds(nbase, SLICE)])

    pltpu.sync_copy(deg_i_sp.at[pl.ds(nbase, SLICE)], nb)

    def prep_i(i, _):
      s = pl.ds(i * LANES, LANES)
      nb[s] = _rsqrt(jnp.maximum(nb[s], 1.0))
      return 0
    lax.fori_loop(0, VPT, prep_i, 0)
    pltpu.sync_copy(nb, deg_i_sp.at[pl.ds(nbase, SLICE)])
    plsc.subcore_barrier()

    for layer in range(L):
      def edge_chunk(g, _):
        off = ebase + g * CH
        pltpu.sync_copy(src_hbm.at[pl.ds(off, CH)], src_buf)
        pltpu.sync_copy(p_sp.at[src_buf], val_buf)
        pltpu.sync_copy(dst_hbm.at[pl.ds(off, CH)], dst_buf)
        pltpu.sync_copy(val_buf, agg_sp.at[dst_buf], add=True)
        return 0
      lax.fori_loop(0, CHUNKS, edge_chunk, 0)
      plsc.subcore_barrier()

      pltpu.sync_copy(agg_sp.at[pl.ds(nbase, SLICE)], agg_loc)
      pltpu.sync_copy(deg_i_sp.at[pl.ds(nbase, SLICE)], na)
      pltpu.sync_copy(deg_o_sp.at[pl.ds(nbase, SLICE)], nb)

      def node(i, c):
        s = pl.ds(i * LANES, LANES)
        hn = agg_loc[s] * na[s]
        np_[s] = hn * nb[s]
        agg_loc[s] = zeros16
        return c + hn * hn
      c = lax.fori_loop(0, VPT, node, zeros16)
      cbuf[...] = c
      pltpu.sync_copy(cbuf, c_sp.at[pl.ds(sid * LANES, LANES)])
      pltpu.sync_copy(np_, p_sp.at[pl.ds(nbase, SLICE)])
      pltpu.sync_copy(agg_loc, agg_sp.at[pl.ds(nbase, SLICE)])
      plsc.subcore_barrier()

      @pl.when(sid == 0)
      def _reduce():
        pltpu.sync_copy(c_sp, red_buf)
        acc = zeros16
        for r in range(TILES):
          acc = acc + red_buf[pl.ds(r * LANES, LANES)]
        cbuf[...] = acc
        pltpu.sync_copy(cbuf, out_hbm.at[pl.ds(layer * LANES, LANES)])


@functools.partial(jax.jit, static_argnums=())
def _sc_call(h1, src1d, dst1d):
  mesh = plsc.VectorSubcoreMesh(core_axis_name="c", subcore_axis_name="s")
  f = pl.kernel(
      _sc_body,
      out_type=jax.ShapeDtypeStruct((L * LANES,), jnp.float32),
      mesh=mesh,
      scratch_types=[
          pltpu.VMEM_SHARED((NP,), jnp.float32),
          pltpu.VMEM_SHARED((NP,), jnp.float32),
          pltpu.VMEM_SHARED((NP,), jnp.float32),
          pltpu.VMEM_SHARED((NP,), jnp.float32),
          pltpu.VMEM_SHARED((TILES * LANES,), jnp.float32),
          pltpu.VMEM((CH,), jnp.int32),
          pltpu.VMEM((CH,), jnp.int32),
          pltpu.VMEM((CH,), jnp.float32),
          pltpu.VMEM((CH,), jnp.float32),
          pltpu.VMEM((SLICE,), jnp.float32),
          pltpu.VMEM((SLICE,), jnp.float32),
          pltpu.VMEM((SLICE,), jnp.float32),
          pltpu.VMEM((SLICE,), jnp.float32),
          pltpu.VMEM((LANES,), jnp.float32),
          pltpu.VMEM((TILES * LANES,), jnp.float32),
      ],
  )
  return f(h1, src1d, dst1d)


def kernel(h, edge_index):
  h1 = jnp.pad(h[:, 0], (0, NP - N))
  out = _sc_call(h1, edge_index[0], edge_index[1])
  return jnp.sum(out.reshape(L, LANES), axis=1)
